# baseline (speedup 1.0000x reference)
import jax
import jax.numpy as jnp
from jax import lax
from jax.experimental import pallas as pl
from jax.experimental.pallas import tpu as pltpu
from jax.experimental.pallas import tpu_sc as plsc

_N = 10000
_DEG = 32
_E = _N * _DEG
_H = 128
_HID = 64
_K = 4

_CW = 128
_NCHUNK = _E // _CW
_NW = 32
_NCW = -(-_NCHUNK // _NW)
_BN = 200


def _precompute_body(h_ref, w1t_ref, b1_ref, a2_ref, b_ref):
    ab = jnp.dot(h_ref[...].astype(jnp.bfloat16), w1t_ref[...].astype(jnp.bfloat16),
                 preferred_element_type=jnp.float32)
    aa = ab[:, :_HID] + b1_ref[...]
    a2_ref[...] = jnp.concatenate([aa, aa], axis=1)
    b_ref[...] = ab[:, _HID:]


def _precompute(h, w1t, b1):
    return pl.pallas_call(
        _precompute_body,
        out_shape=[
            jax.ShapeDtypeStruct((_N, 2 * _HID), jnp.float32),
            jax.ShapeDtypeStruct((_N, _HID), jnp.float32),
        ],
    )(h, w1t, b1)


def _gather_body(b_hbm, idx_hbm, out_hbm, idx_v, rows0, rows1, sem0, sem1):
    wid = lax.axis_index("s") * 2 + lax.axis_index("c")
    base = wid * _NCW
    ncw = jnp.minimum(_NCW, _NCHUNK - base)
    pltpu.sync_copy(idx_hbm.at[pl.ds(base, _NCW)], idx_v)
    pltpu.make_async_copy(b_hbm.at[idx_v.at[0]], rows0, sem0).start()

    def pair(jj, carry):
        c0 = 2 * jj
        c1 = c0 + 1

        @pl.when(c0 < ncw)
        def _():
            @pl.when(c1 < ncw)
            def _():
                pltpu.make_async_copy(b_hbm.at[idx_v.at[c1]], rows1, sem1).start()

            pltpu.make_async_copy(b_hbm.at[idx_v.at[c0]], rows0, sem0).wait()
            pltpu.sync_copy(rows0, out_hbm.at[base + c0])

        @pl.when(c1 < ncw)
        def _():
            @pl.when(c1 + 1 < ncw)
            def _():
                pltpu.make_async_copy(b_hbm.at[idx_v.at[c1 + 1]], rows0, sem0).start()

            pltpu.make_async_copy(b_hbm.at[idx_v.at[c1]], rows1, sem1).wait()
            pltpu.sync_copy(rows1, out_hbm.at[base + c1])

        return carry

    lax.fori_loop(0, (_NCW + 1) // 2, pair, 0)


def _gather(b, idx2d):
    fn = pl.kernel(
        _gather_body,
        out_type=jax.ShapeDtypeStruct((_NCHUNK, _CW, _HID), jnp.float32),
        mesh=plsc.VectorSubcoreMesh(core_axis_name="c", subcore_axis_name="s"),
        compiler_params=pltpu.CompilerParams(use_tc_tiling_on_sc=False),
        scratch_types=[
            pltpu.VMEM((_NW * _NCW // _NW, _CW), jnp.int32),
            pltpu.VMEM((_CW, _HID), jnp.float32),
            pltpu.VMEM((_CW, _HID), jnp.float32),
            pltpu.SemaphoreType.DMA,
            pltpu.SemaphoreType.DMA,
        ],
    )
    return fn(b, idx2d)


def _score_body(a2_ref, bg_ref, dst_ref, w2blk_ref, b2_ref, edst_ref, ew_ref):
    a2 = a2_ref[...]
    lo = jnp.zeros((_BN, _DEG), jnp.float32)
    for t in range(_DEG // 2):
        xt = bg_ref[t]
        ht = jnp.maximum(a2 + xt, 0.0).astype(jnp.bfloat16)
        wt = w2blk_ref[t * 2 * _HID:(t + 1) * 2 * _HID, :].astype(jnp.bfloat16)
        lo = lo + jnp.dot(ht, wt, preferred_element_type=jnp.float32)
    s = jax.nn.sigmoid(lo + b2_ref[0])
    iota = lax.broadcasted_iota(jnp.int32, (_BN, _DEG), 1)
    dsts = dst_ref[...]
    d_cols = []
    w_cols = []
    neg_inf = jnp.float32(-jnp.inf)
    for _ in range(_K):
        m = jnp.max(s, axis=1, keepdims=True)
        pos = jnp.min(jnp.where(s == m, iota, _DEG), axis=1, keepdims=True)
        sel = iota == pos
        d_cols.append(jnp.sum(jnp.where(sel, dsts, 0), axis=1, keepdims=True))
        w_cols.append(m)
        s = jnp.where(sel, neg_inf, s)
    edst_ref[...] = jnp.concatenate(d_cols, axis=1)
    ew_ref[...] = jnp.concatenate(w_cols, axis=1)


def _score(a2, bg2, dst2d, w2blk, b2):
    grid = (_N // _BN,)
    return pl.pallas_call(
        _score_body,
        grid=grid,
        in_specs=[
            pl.BlockSpec((_BN, 2 * _HID), lambda i: (i, 0)),
            pl.BlockSpec((_DEG // 2, _BN, 2 * _HID), lambda i: (i, 0, 0)),
            pl.BlockSpec((_BN, _DEG), lambda i: (i, 0)),
            pl.BlockSpec((_DEG * _HID, _DEG), lambda i: (0, 0)),
            pl.BlockSpec(memory_space=pltpu.SMEM),
        ],
        out_specs=[
            pl.BlockSpec((_BN, _K), lambda i: (i, 0)),
            pl.BlockSpec((_BN, _K), lambda i: (i, 0)),
        ],
        out_shape=[
            jax.ShapeDtypeStruct((_N, _K), jnp.int32),
            jax.ShapeDtypeStruct((_N, _K), jnp.float32),
        ],
    )(a2, bg2, dst2d, w2blk, b2)


def kernel(h, src, dst, W1, b1, W2, b2):
    w1t = jnp.concatenate([W1[:, :_H].T, W1[:, _H:].T], axis=1)
    a2, b = _precompute(h, w1t, b1.reshape(1, _HID))
    nb = _N // _BN
    idx_perm = (dst.reshape(nb, _BN, _DEG // 2, 2)
                .transpose(0, 2, 1, 3).reshape(_NCHUNK, _CW))
    idx2d = jnp.concatenate(
        [idx_perm, jnp.zeros((_NW * _NCW - _NCHUNK, _CW), jnp.int32)], axis=0)
    bg = _gather(b, idx2d)
    bg2 = bg.reshape(nb * _DEG // 2, _BN, 2 * _HID)
    w2blk = jnp.kron(jnp.eye(_DEG, dtype=jnp.float32), W2.T)
    edst, ew = _score(a2, bg2, dst.reshape(_N, _DEG), w2blk, b2)
    esrc = jnp.repeat(jnp.arange(_N, dtype=jnp.int32), _K)
    edge_index = jnp.stack([esrc, edst.reshape(-1)], axis=0)
    return edge_index, ew.reshape(-1)


# speedup vs baseline: 1.6518x; 1.6518x over previous
import jax
import jax.numpy as jnp
from jax import lax
from jax.experimental import pallas as pl
from jax.experimental.pallas import tpu as pltpu
from jax.experimental.pallas import tpu_sc as plsc

_N = 10000
_DEG = 32
_E = _N * _DEG
_H = 128
_HID = 64
_K = 4

_CW = 128
_NCHUNK = _E // _CW
_NW = 32
_NCW = -(-_NCHUNK // _NW)
_BN = 200


def _precompute_body(h_ref, w1t_ref, b1_ref, a2_ref, b_ref):
    ab = jnp.dot(h_ref[...].astype(jnp.bfloat16), w1t_ref[...].astype(jnp.bfloat16),
                 preferred_element_type=jnp.float32)
    aa = ab[:, :_HID] + b1_ref[...]
    a2_ref[...] = jnp.concatenate([aa, aa], axis=1)
    b_ref[...] = ab[:, _HID:]


def _precompute(h, w1t, b1):
    return pl.pallas_call(
        _precompute_body,
        out_shape=[
            jax.ShapeDtypeStruct((_N, 2 * _HID), jnp.float32),
            jax.ShapeDtypeStruct((_N, _HID), jnp.float32),
        ],
    )(h, w1t, b1)


def _gather_body(b_hbm, idx_hbm, out_hbm, idx_v, rows0, rows1, sem0, sem1):
    wid = lax.axis_index("s") * 2 + lax.axis_index("c")
    base = wid * _NCW
    ncw = jnp.minimum(_NCW, _NCHUNK - base)
    pltpu.sync_copy(idx_hbm.at[pl.ds(base, _NCW)], idx_v)
    pltpu.make_async_copy(b_hbm.at[idx_v.at[0]], rows0, sem0).start()

    def pair(jj, carry):
        c0 = 2 * jj
        c1 = c0 + 1

        @pl.when(c0 < ncw)
        def _():
            @pl.when(c1 < ncw)
            def _():
                pltpu.make_async_copy(b_hbm.at[idx_v.at[c1]], rows1, sem1).start()

            pltpu.make_async_copy(b_hbm.at[idx_v.at[c0]], rows0, sem0).wait()
            pltpu.sync_copy(rows0, out_hbm.at[base + c0])

        @pl.when(c1 < ncw)
        def _():
            @pl.when(c1 + 1 < ncw)
            def _():
                pltpu.make_async_copy(b_hbm.at[idx_v.at[c1 + 1]], rows0, sem0).start()

            pltpu.make_async_copy(b_hbm.at[idx_v.at[c1]], rows1, sem1).wait()
            pltpu.sync_copy(rows1, out_hbm.at[base + c1])

        return carry

    lax.fori_loop(0, (_NCW + 1) // 2, pair, 0)


def _gather(b, idx2d):
    fn = pl.kernel(
        _gather_body,
        out_type=jax.ShapeDtypeStruct((_NCHUNK, _CW, _HID), jnp.float32),
        mesh=plsc.VectorSubcoreMesh(core_axis_name="c", subcore_axis_name="s"),
        compiler_params=pltpu.CompilerParams(use_tc_tiling_on_sc=False),
        scratch_types=[
            pltpu.VMEM((_NW * _NCW // _NW, _CW), jnp.int32),
            pltpu.VMEM((_CW, _HID), jnp.float32),
            pltpu.VMEM((_CW, _HID), jnp.float32),
            pltpu.SemaphoreType.DMA,
            pltpu.SemaphoreType.DMA,
        ],
    )
    return fn(b, idx2d)


def _score_body(a2_ref, bg_ref, dst_ref, w2blk_ref, b2_ref, edst_ref, ew_ref):
    a2 = a2_ref[...]
    lo = jnp.zeros((_BN, _DEG), jnp.float32)
    for t in range(_DEG // 2):
        xt = bg_ref[t]
        ht = jnp.maximum(a2 + xt, 0.0).astype(jnp.bfloat16)
        wt = w2blk_ref[t * 2 * _HID:(t + 1) * 2 * _HID, :].astype(jnp.bfloat16)
        lo = lo + jnp.dot(ht, wt, preferred_element_type=jnp.float32)
    s = jax.nn.sigmoid(lo + b2_ref[0])
    iota = lax.broadcasted_iota(jnp.int32, (_BN, _DEG), 1)
    dsts = dst_ref[...]
    d_cols = []
    w_cols = []
    neg_inf = jnp.float32(-jnp.inf)
    for _ in range(_K):
        m = jnp.max(s, axis=1, keepdims=True)
        pos = jnp.min(jnp.where(s == m, iota, _DEG), axis=1, keepdims=True)
        sel = iota == pos
        d_cols.append(jnp.sum(jnp.where(sel, dsts, 0), axis=1, keepdims=True))
        w_cols.append(m)
        s = jnp.where(sel, neg_inf, s)
    edst_ref[...] = jnp.concatenate(d_cols, axis=1)
    ew_ref[...] = jnp.concatenate(w_cols, axis=1)


def _score(a2, bg2, dst2d, w2blk, b2):
    grid = (_N // _BN,)
    return pl.pallas_call(
        _score_body,
        grid=grid,
        in_specs=[
            pl.BlockSpec((_BN, 2 * _HID), lambda i: (i, 0)),
            pl.BlockSpec((_DEG // 2, _BN, 2 * _HID), lambda i: (i, 0, 0)),
            pl.BlockSpec((_BN, _DEG), lambda i: (i, 0)),
            pl.BlockSpec((_DEG * _HID, _DEG), lambda i: (0, 0)),
            pl.BlockSpec(memory_space=pltpu.SMEM),
        ],
        out_specs=[
            pl.BlockSpec((_BN, _K), lambda i: (i, 0)),
            pl.BlockSpec((_BN, _K), lambda i: (i, 0)),
        ],
        out_shape=[
            jax.ShapeDtypeStruct((_N, _K), jnp.int32),
            jax.ShapeDtypeStruct((_N, _K), jnp.float32),
        ],
    )(a2, bg2, dst2d, w2blk, b2)


def kernel(h, src, dst, W1, b1, W2, b2):
    w1t = jnp.concatenate([W1[:, :_H].T, W1[:, _H:].T], axis=1)
    a2, b = _precompute(h, w1t, b1.reshape(1, _HID))
    nb = _N // _BN
    k = jnp.arange(_E, dtype=jnp.int32)
    p = k & 1
    r = k >> 1
    n = r % _BN
    t = (r // _BN) % (_DEG // 2)
    i = r // (_BN * _DEG // 2)
    perm = ((i * _BN + n) * _DEG + 2 * t + p)
    idx_perm = jnp.take(dst, perm).reshape(_NCHUNK, _CW)
    idx2d = jnp.concatenate(
        [idx_perm, jnp.zeros((_NW * _NCW - _NCHUNK, _CW), jnp.int32)], axis=0)
    bg = _gather(b, idx2d)
    bg2 = bg.reshape(nb * _DEG // 2, _BN, 2 * _HID)
    w2blk = jnp.kron(jnp.eye(_DEG, dtype=jnp.float32), W2.T)
    edst, ew = _score(a2, bg2, dst.reshape(_N, _DEG), w2blk, b2)
    esrc = jnp.repeat(jnp.arange(_N, dtype=jnp.int32), _K)
    edge_index = jnp.stack([esrc, edst.reshape(-1)], axis=0)
    return edge_index, ew.reshape(-1)


# speedup vs baseline: 1.9735x; 1.1947x over previous
import functools

import jax
import jax.numpy as jnp
from jax import lax
from jax.experimental import pallas as pl
from jax.experimental.pallas import tpu as pltpu
from jax.experimental.pallas import tpu_sc as plsc

_N = 10000
_DEG = 32
_E = _N * _DEG
_H = 128
_HID = 64
_K = 4

_CW = 128
_NW = 32
_BN = 200
_P = 2


def _precompute_body(h_ref, w1t_ref, b1_ref, a2_ref, b_ref):
    ab = jnp.dot(h_ref[...].astype(jnp.bfloat16), w1t_ref[...].astype(jnp.bfloat16),
                 preferred_element_type=jnp.float32)
    aa = ab[:, :_HID] + b1_ref[...]
    a2_ref[...] = jnp.concatenate([aa, aa], axis=1)
    b_ref[...] = ab[:, _HID:]


def _precompute(h, w1t, b1):
    return pl.pallas_call(
        _precompute_body,
        out_shape=[
            jax.ShapeDtypeStruct((_N, 2 * _HID), jnp.float32),
            jax.ShapeDtypeStruct((_N, _HID), jnp.float32),
        ],
    )(h, w1t, b1)


def _gather_body(nchunk, ncw_max, b_hbm, idx_hbm, out_hbm, idx_v, rows0, rows1, sem0, sem1):
    wid = lax.axis_index("s") * 2 + lax.axis_index("c")
    base = wid * ncw_max
    ncw = jnp.minimum(ncw_max, nchunk - base)
    pltpu.sync_copy(idx_hbm.at[pl.ds(base, ncw_max)], idx_v)
    pltpu.make_async_copy(b_hbm.at[idx_v.at[0]], rows0, sem0).start()

    def pair(jj, carry):
        c0 = 2 * jj
        c1 = c0 + 1

        @pl.when(c0 < ncw)
        def _():
            @pl.when(c1 < ncw)
            def _():
                pltpu.make_async_copy(b_hbm.at[idx_v.at[c1]], rows1, sem1).start()

            pltpu.make_async_copy(b_hbm.at[idx_v.at[c0]], rows0, sem0).wait()
            pltpu.sync_copy(rows0, out_hbm.at[base + c0])

        @pl.when(c1 < ncw)
        def _():
            @pl.when(c1 + 1 < ncw)
            def _():
                pltpu.make_async_copy(b_hbm.at[idx_v.at[c1 + 1]], rows0, sem0).start()

            pltpu.make_async_copy(b_hbm.at[idx_v.at[c1]], rows1, sem1).wait()
            pltpu.sync_copy(rows1, out_hbm.at[base + c1])

        return carry

    lax.fori_loop(0, (ncw_max + 1) // 2, pair, 0)


def _gather(b, idx2d, nchunk):
    ncw_max = -(-nchunk // _NW)
    fn = pl.kernel(
        functools.partial(_gather_body, nchunk, ncw_max),
        out_type=jax.ShapeDtypeStruct((nchunk, _CW, _HID), jnp.float32),
        mesh=plsc.VectorSubcoreMesh(core_axis_name="c", subcore_axis_name="s"),
        compiler_params=pltpu.CompilerParams(use_tc_tiling_on_sc=False),
        scratch_types=[
            pltpu.VMEM((ncw_max, _CW), jnp.int32),
            pltpu.VMEM((_CW, _HID), jnp.float32),
            pltpu.VMEM((_CW, _HID), jnp.float32),
            pltpu.SemaphoreType.DMA,
            pltpu.SemaphoreType.DMA,
        ],
    )
    return fn(b, idx2d)


def _score_body(a2_ref, bg_ref, dst_ref, w2blk_ref, b2_ref, edst_ref, ew_ref):
    a2 = a2_ref[...]
    lo = jnp.zeros((_BN, _DEG), jnp.float32)
    for t in range(_DEG // 2):
        xt = bg_ref[:, t, :]
        ht = jnp.maximum(a2 + xt, 0.0).astype(jnp.bfloat16)
        wt = w2blk_ref[t * 2 * _HID:(t + 1) * 2 * _HID, :].astype(jnp.bfloat16)
        lo = lo + jnp.dot(ht, wt, preferred_element_type=jnp.float32)
    s = jax.nn.sigmoid(lo + b2_ref[0])
    iota = lax.broadcasted_iota(jnp.int32, (_BN, _DEG), 1)
    dsts = dst_ref[...]
    d_cols = []
    w_cols = []
    neg_inf = jnp.float32(-jnp.inf)
    for _ in range(_K):
        m = jnp.max(s, axis=1, keepdims=True)
        pos = jnp.min(jnp.where(s == m, iota, _DEG), axis=1, keepdims=True)
        sel = iota == pos
        d_cols.append(jnp.sum(jnp.where(sel, dsts, 0), axis=1, keepdims=True))
        w_cols.append(m)
        s = jnp.where(sel, neg_inf, s)
    edst_ref[...] = jnp.concatenate(d_cols, axis=1)
    ew_ref[...] = jnp.concatenate(w_cols, axis=1)


def _score(a2, bg2, dst2d, w2blk, b2, n_nodes):
    grid = (n_nodes // _BN,)
    return pl.pallas_call(
        _score_body,
        grid=grid,
        in_specs=[
            pl.BlockSpec((_BN, 2 * _HID), lambda i: (i, 0)),
            pl.BlockSpec((_BN, _DEG // 2, 2 * _HID), lambda i: (i, 0, 0)),
            pl.BlockSpec((_BN, _DEG), lambda i: (i, 0)),
            pl.BlockSpec((_DEG * _HID, _DEG), lambda i: (0, 0)),
            pl.BlockSpec(memory_space=pltpu.SMEM),
        ],
        out_specs=[
            pl.BlockSpec((_BN, _K), lambda i: (i, 0)),
            pl.BlockSpec((_BN, _K), lambda i: (i, 0)),
        ],
        out_shape=[
            jax.ShapeDtypeStruct((n_nodes, _K), jnp.int32),
            jax.ShapeDtypeStruct((n_nodes, _K), jnp.float32),
        ],
    )(a2, bg2, dst2d, w2blk, b2)


def kernel(h, src, dst, W1, b1, W2, b2):
    w1t = jnp.concatenate([W1[:, :_H].T, W1[:, _H:].T], axis=1)
    a2, b = _precompute(h, w1t, b1.reshape(1, _HID))
    w2blk = jnp.kron(jnp.eye(_DEG, dtype=jnp.float32), W2.T)
    dst2d = dst.reshape(_N, _DEG)
    np_ = _N // _P
    nch = np_ * _DEG // _CW
    ncw_max = -(-nch // _NW)
    idx3 = dst.reshape(_P, nch, _CW)
    pad = jnp.zeros((_NW * ncw_max - nch, _CW), jnp.int32)
    bgs = [_gather(b, jnp.concatenate([idx3[p], pad], axis=0), nch)
           for p in range(_P)]
    edst_l, ew_l = [], []
    for p in range(_P):
        bg2 = bgs[p].reshape(np_, _DEG // 2, 2 * _HID)
        e_p, w_p = _score(
            lax.slice_in_dim(a2, p * np_, (p + 1) * np_, axis=0), bg2,
            lax.slice_in_dim(dst2d, p * np_, (p + 1) * np_, axis=0),
            w2blk, b2, np_)
        edst_l.append(e_p)
        ew_l.append(w_p)
    edst = jnp.concatenate(edst_l, axis=0)
    ew = jnp.concatenate(ew_l, axis=0)
    esrc = jnp.repeat(jnp.arange(_N, dtype=jnp.int32), _K)
    edge_index = jnp.stack([esrc, edst.reshape(-1)], axis=0)
    return edge_index, ew.reshape(-1)
